# baseline (speedup 1.0000x reference)
import functools

import jax
import jax.numpy as jnp
from jax import lax
from jax.experimental import pallas as pl
from jax.experimental.pallas import tpu as pltpu
from jax.experimental.pallas import tpu_sc as plsc

N_NODES = 10000
N_ROWS = 80
N_PAD = N_ROWS * 128
E_EDGES = 320000
E_COLS = E_EDGES // 128
NUM_CORES = 2
NUM_SUBCORES = 16
NUM_WORKERS = NUM_CORES * NUM_SUBCORES
E_PER_WORKER = E_EDGES // NUM_WORKERS
WINDOW_COLS = E_COLS // NUM_WORKERS + 1
WINDOW = WINDOW_COLS * 128
LANES = 16
WEIGHT_CONST = 0.01


def _sc_body(phi_hbm, pos_hbm, ei_hbm, out_hbm,
             phi_v, pos_v, src_v, dst_v, lap_v, deg_v, rows_v,
             phi_sh, lap_sh, deg_sh, sem):
    cid = lax.axis_index("c")
    sid = lax.axis_index("s")
    wid = sid * NUM_CORES + cid
    base = wid * E_PER_WORKER
    col0 = (wid * E_COLS) // NUM_WORKERS
    off = base - col0 * 128

    with jax.named_scope("stage_shared"):
        @pl.when(sid == 0)
        def _():
            pltpu.sync_copy(phi_hbm, phi_sh)

    c1 = pltpu.async_copy(pos_hbm, pos_v, sem)
    c2 = pltpu.async_copy(
        ei_hbm.at[0, pl.ds(col0 * 128, WINDOW)], src_v, sem)
    c3 = pltpu.async_copy(
        ei_hbm.at[1, pl.ds(col0 * 128, WINDOW)], dst_v, sem)

    zeros = jnp.zeros((LANES,), jnp.float32)
    lane16 = jnp.arange(LANES, dtype=jnp.int32)

    with jax.named_scope("zero"):
        for k in range(N_ROWS // LANES):
            rows_v[pl.ds(k * LANES, LANES)] = lane16 + (k * LANES)

        @plsc.parallel_loop(0, N_PAD // LANES, unroll=8)
        def _zero(j):
            r = j >> 3
            c = (j & 7) * LANES
            lap_v[r, pl.ds(c, LANES)] = zeros
            deg_v[r, pl.ds(c, LANES)] = zeros

    with jax.named_scope("zero_shared"):
        @pl.when(sid == 0)
        def _():
            pltpu.sync_copy(lap_v, lap_sh)
            pltpu.sync_copy(deg_v, deg_sh)

    plsc.subcore_barrier()

    with jax.named_scope("stage_local"):
        pltpu.sync_copy(phi_sh, phi_v)
        c1.wait()
        c2.wait()
        c3.wait()

    ones = jnp.ones((LANES,), jnp.float32)

    with jax.named_scope("edges"):
        @plsc.parallel_loop(0, E_PER_WORKER // LANES, unroll=16)
        def _edges(i):
            o = off + i * LANES
            s = src_v[pl.ds(o, LANES)]
            d = dst_v[pl.ds(o, LANES)]
            dphi = plsc.load_gather(phi_v, [s]) - plsc.load_gather(phi_v, [d])
            s3 = s * 3
            d3 = d * 3
            dx = plsc.load_gather(pos_v, [s3]) - plsc.load_gather(pos_v, [d3])
            dy = plsc.load_gather(pos_v, [s3 + 1]) - plsc.load_gather(pos_v, [d3 + 1])
            dz = plsc.load_gather(pos_v, [s3 + 2]) - plsc.load_gather(pos_v, [d3 + 2])
            dist2 = dx * dx + dy * dy + dz * dz + 1e-8
            dhi = d >> 7
            dlo = d & 127
            plsc.addupdate_scatter(lap_v, [dhi, dlo], dphi / dist2)
            plsc.addupdate_scatter(deg_v, [dhi, dlo], ones)

    with jax.named_scope("merge"):
        pltpu.sync_copy(lap_v, lap_sh.at[rows_v], add=True)
        pltpu.sync_copy(deg_v, deg_sh.at[rows_v], add=True)

    plsc.subcore_barrier()

    with jax.named_scope("out"):
        @pl.when(sid == 0)
        def _():
            o0 = pltpu.async_copy(lap_sh, out_hbm.at[cid], sem)
            o1 = pltpu.async_copy(deg_sh, out_hbm.at[NUM_CORES + cid], sem)
            o0.wait()
            o1.wait()


_sc_partials = functools.partial(
    pl.kernel,
    out_type=jax.ShapeDtypeStruct((2 * NUM_CORES, N_ROWS, 128), jnp.float32),
    mesh=plsc.VectorSubcoreMesh(core_axis_name="c", subcore_axis_name="s",
                                num_cores=NUM_CORES,
                                num_subcores=NUM_SUBCORES),
    compiler_params=pltpu.CompilerParams(needs_layout_passes=False),
    scratch_types=[
        pltpu.VMEM((N_NODES,), jnp.float32),
        pltpu.VMEM((3 * N_NODES,), jnp.float32),
        pltpu.VMEM((WINDOW,), jnp.int32),
        pltpu.VMEM((WINDOW,), jnp.int32),
        pltpu.VMEM((N_ROWS, 128), jnp.float32),
        pltpu.VMEM((N_ROWS, 128), jnp.float32),
        pltpu.VMEM((N_ROWS,), jnp.int32),
        pltpu.VMEM_SHARED((N_NODES,), jnp.float32),
        pltpu.VMEM_SHARED((N_ROWS, 128), jnp.float32),
        pltpu.VMEM_SHARED((N_ROWS, 128), jnp.float32),
        pltpu.SemaphoreType.DMA,
    ],
)(_sc_body)


def _finalize_body(part_ref, out_ref):
    part = part_ref[...]
    lap = part[0] + part[1]
    deg = part[2] + part[3]
    curv = lap / (deg + 1e-8)
    out_ref[0, 0] = WEIGHT_CONST * jnp.sum(curv * curv) / float(N_NODES)


def kernel(x, pos, edge_index):
    phi = x[:, 8]
    pos_flat = jnp.reshape(pos, (-1,))

    part = _sc_partials(phi, pos_flat, edge_index)

    loss = pl.pallas_call(
        _finalize_body,
        out_shape=jax.ShapeDtypeStruct((1, 1), jnp.float32),
        in_specs=[pl.BlockSpec((2 * NUM_CORES, N_ROWS, 128), lambda: (0, 0, 0))],
        out_specs=pl.BlockSpec(memory_space=pltpu.SMEM),
    )(part)
    return jnp.reshape(loss, ())


# speedup vs baseline: 1.2553x; 1.2553x over previous
import functools

import jax
import jax.numpy as jnp
from jax import lax
from jax.experimental import pallas as pl
from jax.experimental.pallas import tpu as pltpu
from jax.experimental.pallas import tpu_sc as plsc

N_NODES = 10000
E_EDGES = 320000
E_COLS = E_EDGES // 128
NUM_CORES = 2
NUM_SUBCORES = 16
NUM_WORKERS = NUM_CORES * NUM_SUBCORES
E_PER_WORKER = E_EDGES // NUM_WORKERS
WINDOW_COLS = E_COLS // NUM_WORKERS + 1
WINDOW = WINDOW_COLS * 128
LANES = 16
WEIGHT_CONST = 0.01


def _sc_body(phi_hbm, pos_hbm, ei_hbm, out_hbm,
             phi_v, pos_v, src_v, dst_v, lap_v, deg_v,
             phi_sh, pos_sh, sem):
    cid = lax.axis_index("c")
    sid = lax.axis_index("s")
    wid = sid * NUM_CORES + cid
    base = wid * E_PER_WORKER
    col0 = (wid * E_COLS) // NUM_WORKERS
    off = base - col0 * 128

    with jax.named_scope("stage_shared"):
        @pl.when(sid == 0)
        def _():
            pltpu.sync_copy(phi_hbm, phi_sh)

        @pl.when(sid == 1)
        def _():
            pltpu.sync_copy(pos_hbm, pos_sh)

    c2 = pltpu.async_copy(
        ei_hbm.at[0, pl.ds(col0 * 128, WINDOW)], src_v, sem)
    c3 = pltpu.async_copy(
        ei_hbm.at[1, pl.ds(col0 * 128, WINDOW)], dst_v, sem)

    zeros = jnp.zeros((LANES,), jnp.float32)

    with jax.named_scope("zero"):
        @plsc.parallel_loop(0, N_NODES // LANES, unroll=8)
        def _zero(j):
            lap_v[pl.ds(j * LANES, LANES)] = zeros
            deg_v[pl.ds(j * LANES, LANES)] = zeros

    plsc.subcore_barrier()

    with jax.named_scope("stage_local"):
        pltpu.sync_copy(phi_sh, phi_v)
        pltpu.sync_copy(pos_sh, pos_v)
        c2.wait()
        c3.wait()

    ones = jnp.ones((LANES,), jnp.float32)

    with jax.named_scope("edges"):
        @plsc.parallel_loop(off, off + E_PER_WORKER, step=LANES, unroll=8)
        def _edges(o):
            s = src_v[pl.ds(o, LANES)]
            d = dst_v[pl.ds(o, LANES)]
            dphi = plsc.load_gather(phi_v, [s]) - plsc.load_gather(phi_v, [d])
            dx = plsc.load_gather(pos_v, [s]) - plsc.load_gather(pos_v, [d])
            dy = (plsc.load_gather(pos_v, [s + N_NODES])
                  - plsc.load_gather(pos_v, [d + N_NODES]))
            dz = (plsc.load_gather(pos_v, [s + 2 * N_NODES])
                  - plsc.load_gather(pos_v, [d + 2 * N_NODES]))
            dist2 = dx * dx + dy * dy + dz * dz + 1e-8
            plsc.addupdate_scatter(lap_v, [d], dphi / dist2)
            plsc.addupdate_scatter(deg_v, [d], ones)

    with jax.named_scope("out"):
        o0 = pltpu.async_copy(lap_v, out_hbm.at[wid], sem)
        o1 = pltpu.async_copy(deg_v, out_hbm.at[NUM_WORKERS + wid], sem)
        o0.wait()
        o1.wait()


_sc_partials = functools.partial(
    pl.kernel,
    out_type=jax.ShapeDtypeStruct((2 * NUM_WORKERS, N_NODES), jnp.float32),
    mesh=plsc.VectorSubcoreMesh(core_axis_name="c", subcore_axis_name="s",
                                num_cores=NUM_CORES,
                                num_subcores=NUM_SUBCORES),
    compiler_params=pltpu.CompilerParams(needs_layout_passes=False),
    scratch_types=[
        pltpu.VMEM((N_NODES,), jnp.float32),
        pltpu.VMEM((3 * N_NODES,), jnp.float32),
        pltpu.VMEM((WINDOW,), jnp.int32),
        pltpu.VMEM((WINDOW,), jnp.int32),
        pltpu.VMEM((N_NODES,), jnp.float32),
        pltpu.VMEM((N_NODES,), jnp.float32),
        pltpu.VMEM_SHARED((N_NODES,), jnp.float32),
        pltpu.VMEM_SHARED((3 * N_NODES,), jnp.float32),
        pltpu.SemaphoreType.DMA,
    ],
)(_sc_body)


def _finalize_body(part_ref, out_ref):
    part = part_ref[...]
    lap = jnp.sum(part[:NUM_WORKERS, :], axis=0)
    deg = jnp.sum(part[NUM_WORKERS:, :], axis=0)
    curv = lap / (deg + 1e-8)
    out_ref[0, 0] = WEIGHT_CONST * jnp.sum(curv * curv) / float(N_NODES)


def kernel(x, pos, edge_index):
    phi = x[:, 8]
    pos_t = jnp.reshape(jnp.swapaxes(pos, 0, 1), (-1,))

    part = _sc_partials(phi, pos_t, edge_index)

    loss = pl.pallas_call(
        _finalize_body,
        out_shape=jax.ShapeDtypeStruct((1, 1), jnp.float32),
        in_specs=[pl.BlockSpec((2 * NUM_WORKERS, N_NODES), lambda: (0, 0))],
        out_specs=pl.BlockSpec(memory_space=pltpu.SMEM),
    )(part)
    return jnp.reshape(loss, ())


# speedup vs baseline: 1.2574x; 1.0016x over previous
import functools

import jax
import jax.numpy as jnp
from jax import lax
from jax.experimental import pallas as pl
from jax.experimental.pallas import tpu as pltpu
from jax.experimental.pallas import tpu_sc as plsc

N_NODES = 10000
E_EDGES = 320000
E_COLS = E_EDGES // 128
NUM_CORES = 2
NUM_SUBCORES = 16
NUM_WORKERS = NUM_CORES * NUM_SUBCORES
E_PER_WORKER = E_EDGES // NUM_WORKERS
WINDOW_COLS = E_COLS // NUM_WORKERS + 1
WINDOW = WINDOW_COLS * 128
LANES = 16
WEIGHT_CONST = 0.01


def _sc_body(phi_hbm, pos_hbm, ei_hbm, out_hbm,
             phi_v, pos_v, src_v, dst_v, lap_v, deg_v,
             phi_sh, pos_sh, sem):
    cid = lax.axis_index("c")
    sid = lax.axis_index("s")
    wid = sid * NUM_CORES + cid
    base = wid * E_PER_WORKER
    col0 = (wid * E_COLS) // NUM_WORKERS
    off = base - col0 * 128

    with jax.named_scope("stage_shared"):
        @pl.when(sid == 0)
        def _():
            pltpu.sync_copy(phi_hbm, phi_sh)

        @pl.when(sid == 1)
        def _():
            pltpu.sync_copy(pos_hbm, pos_sh)

    c2 = pltpu.async_copy(
        ei_hbm.at[0, pl.ds(col0 * 128, WINDOW)], src_v, sem)
    c3 = pltpu.async_copy(
        ei_hbm.at[1, pl.ds(col0 * 128, WINDOW)], dst_v, sem)

    zeros = jnp.zeros((LANES,), jnp.float32)

    with jax.named_scope("zero"):
        @plsc.parallel_loop(0, N_NODES // LANES, unroll=4)
        def _zero(j):
            lap_v[pl.ds(j * LANES, LANES)] = zeros
            deg_v[pl.ds(j * LANES, LANES)] = zeros

    plsc.subcore_barrier()

    with jax.named_scope("stage_local"):
        pltpu.sync_copy(phi_sh, phi_v)
        pltpu.sync_copy(pos_sh, pos_v)
        c2.wait()
        c3.wait()

    ones = jnp.ones((LANES,), jnp.float32)

    with jax.named_scope("edges"):
        @plsc.parallel_loop(off, off + E_PER_WORKER, step=LANES, unroll=2)
        def _edges(o):
            s = src_v[pl.ds(o, LANES)]
            d = dst_v[pl.ds(o, LANES)]
            dphi = plsc.load_gather(phi_v, [s]) - plsc.load_gather(phi_v, [d])
            dx = plsc.load_gather(pos_v, [s]) - plsc.load_gather(pos_v, [d])
            dy = (plsc.load_gather(pos_v, [s + N_NODES])
                  - plsc.load_gather(pos_v, [d + N_NODES]))
            dz = (plsc.load_gather(pos_v, [s + 2 * N_NODES])
                  - plsc.load_gather(pos_v, [d + 2 * N_NODES]))
            dist2 = dx * dx + dy * dy + dz * dz + 1e-8
            plsc.addupdate_scatter(lap_v, [d], dphi / dist2)
            plsc.addupdate_scatter(deg_v, [d], ones)

    with jax.named_scope("out"):
        o0 = pltpu.async_copy(lap_v, out_hbm.at[wid], sem)
        o1 = pltpu.async_copy(deg_v, out_hbm.at[NUM_WORKERS + wid], sem)
        o0.wait()
        o1.wait()


_sc_partials = functools.partial(
    pl.kernel,
    out_type=jax.ShapeDtypeStruct((2 * NUM_WORKERS, N_NODES), jnp.float32),
    mesh=plsc.VectorSubcoreMesh(core_axis_name="c", subcore_axis_name="s",
                                num_cores=NUM_CORES,
                                num_subcores=NUM_SUBCORES),
    compiler_params=pltpu.CompilerParams(needs_layout_passes=False),
    scratch_types=[
        pltpu.VMEM((N_NODES,), jnp.float32),
        pltpu.VMEM((3 * N_NODES,), jnp.float32),
        pltpu.VMEM((WINDOW,), jnp.int32),
        pltpu.VMEM((WINDOW,), jnp.int32),
        pltpu.VMEM((N_NODES,), jnp.float32),
        pltpu.VMEM((N_NODES,), jnp.float32),
        pltpu.VMEM_SHARED((N_NODES,), jnp.float32),
        pltpu.VMEM_SHARED((3 * N_NODES,), jnp.float32),
        pltpu.SemaphoreType.DMA,
    ],
)(_sc_body)


def _finalize_body(part_ref, out_ref):
    part = part_ref[...]
    lap = jnp.sum(part[:NUM_WORKERS, :], axis=0)
    deg = jnp.sum(part[NUM_WORKERS:, :], axis=0)
    curv = lap / (deg + 1e-8)
    out_ref[0, 0] = WEIGHT_CONST * jnp.sum(curv * curv) / float(N_NODES)


def kernel(x, pos, edge_index):
    phi = x[:, 8]
    pos_t = jnp.reshape(jnp.swapaxes(pos, 0, 1), (-1,))

    part = _sc_partials(phi, pos_t, edge_index)

    loss = pl.pallas_call(
        _finalize_body,
        out_shape=jax.ShapeDtypeStruct((1, 1), jnp.float32),
        in_specs=[pl.BlockSpec((2 * NUM_WORKERS, N_NODES), lambda: (0, 0))],
        out_specs=pl.BlockSpec(memory_space=pltpu.SMEM),
    )(part)
    return jnp.reshape(loss, ())


# speedup vs baseline: 1.2645x; 1.0056x over previous
import functools

import jax
import jax.numpy as jnp
from jax import lax
from jax.experimental import pallas as pl
from jax.experimental.pallas import tpu as pltpu
from jax.experimental.pallas import tpu_sc as plsc

N_NODES = 10000
E_EDGES = 320000
E_COLS = E_EDGES // 128
NUM_CORES = 2
NUM_SUBCORES = 16
NUM_WORKERS = NUM_CORES * NUM_SUBCORES
E_PER_WORKER = E_EDGES // NUM_WORKERS
WINDOW_COLS = E_COLS // NUM_WORKERS + 1
WINDOW = WINDOW_COLS * 128
LANES = 16
WEIGHT_CONST = 0.01


def _sc_body(phi_hbm, pos_hbm, ei_hbm, out_hbm,
             phi_v, pos_v, src_v, dst_v, lap_v, deg_v,
             phi_sh, pos_sh, sem):
    cid = lax.axis_index("c")
    sid = lax.axis_index("s")
    wid = sid * NUM_CORES + cid
    base = wid * E_PER_WORKER
    col0 = (wid * E_COLS) // NUM_WORKERS
    off = base - col0 * 128

    with jax.named_scope("stage_shared"):
        @pl.when(sid == 0)
        def _():
            pltpu.sync_copy(phi_hbm, phi_sh)

        @pl.when(sid == 1)
        def _():
            pltpu.sync_copy(pos_hbm, pos_sh)

    c2 = pltpu.async_copy(
        ei_hbm.at[0, pl.ds(col0 * 128, WINDOW)], src_v, sem)
    c3 = pltpu.async_copy(
        ei_hbm.at[1, pl.ds(col0 * 128, WINDOW)], dst_v, sem)

    zeros = jnp.zeros((LANES,), jnp.float32)

    with jax.named_scope("zero"):
        @plsc.parallel_loop(0, N_NODES // LANES, unroll=8)
        def _zero(j):
            lap_v[pl.ds(j * LANES, LANES)] = zeros
            deg_v[pl.ds(j * LANES, LANES)] = zeros

    plsc.subcore_barrier()

    with jax.named_scope("stage_local"):
        pltpu.sync_copy(phi_sh, phi_v)
        pltpu.sync_copy(pos_sh, pos_v)
        c2.wait()
        c3.wait()

    ones = jnp.ones((LANES,), jnp.float32)

    with jax.named_scope("edges"):
        @plsc.parallel_loop(off, off + E_PER_WORKER, step=LANES, unroll=4)
        def _edges(o):
            s = src_v[pl.ds(o, LANES)]
            d = dst_v[pl.ds(o, LANES)]
            dphi = plsc.load_gather(phi_v, [s]) - plsc.load_gather(phi_v, [d])
            dx = plsc.load_gather(pos_v, [s]) - plsc.load_gather(pos_v, [d])
            dy = (plsc.load_gather(pos_v, [s + N_NODES])
                  - plsc.load_gather(pos_v, [d + N_NODES]))
            dz = (plsc.load_gather(pos_v, [s + 2 * N_NODES])
                  - plsc.load_gather(pos_v, [d + 2 * N_NODES]))
            dist2 = dx * dx + dy * dy + dz * dz + 1e-8
            plsc.addupdate_scatter(lap_v, [d], dphi / dist2)
            plsc.addupdate_scatter(deg_v, [d], ones)

    with jax.named_scope("out"):
        o0 = pltpu.async_copy(lap_v, out_hbm.at[wid], sem)
        o1 = pltpu.async_copy(deg_v, out_hbm.at[NUM_WORKERS + wid], sem)
        o0.wait()
        o1.wait()


_sc_partials = functools.partial(
    pl.kernel,
    out_type=jax.ShapeDtypeStruct((2 * NUM_WORKERS, N_NODES), jnp.float32),
    mesh=plsc.VectorSubcoreMesh(core_axis_name="c", subcore_axis_name="s",
                                num_cores=NUM_CORES,
                                num_subcores=NUM_SUBCORES),
    compiler_params=pltpu.CompilerParams(needs_layout_passes=False),
    scratch_types=[
        pltpu.VMEM((N_NODES,), jnp.float32),
        pltpu.VMEM((3 * N_NODES,), jnp.float32),
        pltpu.VMEM((WINDOW,), jnp.int32),
        pltpu.VMEM((WINDOW,), jnp.int32),
        pltpu.VMEM((N_NODES,), jnp.float32),
        pltpu.VMEM((N_NODES,), jnp.float32),
        pltpu.VMEM_SHARED((N_NODES,), jnp.float32),
        pltpu.VMEM_SHARED((3 * N_NODES,), jnp.float32),
        pltpu.SemaphoreType.DMA,
    ],
)(_sc_body)


def _finalize_body(part_ref, out_ref):
    part = part_ref[...]
    lap = jnp.sum(part[:NUM_WORKERS, :], axis=0)
    deg = jnp.sum(part[NUM_WORKERS:, :], axis=0)
    curv = lap / (deg + 1e-8)
    out_ref[0, 0] = WEIGHT_CONST * jnp.sum(curv * curv) / float(N_NODES)


def kernel(x, pos, edge_index):
    phi = x[:, 8]
    pos_t = jnp.reshape(jnp.swapaxes(pos, 0, 1), (-1,))

    part = _sc_partials(phi, pos_t, edge_index)

    loss = pl.pallas_call(
        _finalize_body,
        out_shape=jax.ShapeDtypeStruct((1, 1), jnp.float32),
        in_specs=[pl.BlockSpec((2 * NUM_WORKERS, N_NODES), lambda: (0, 0))],
        out_specs=pl.BlockSpec(memory_space=pltpu.SMEM),
    )(part)
    return jnp.reshape(loss, ())
